# baseline (speedup 1.0000x reference)
import functools

import jax
import jax.numpy as jnp
from jax.experimental import pallas as pl

B, D, E, H = 32768, 768, 64, 64
BM = 512


def _router_block(x_ref, ah_ref, al_ref, p_ref, o_ref):
    xb = x_ref[...]
    xh = xb.astype(jnp.bfloat16)
    xl = (xb - xh.astype(jnp.float32)).astype(jnp.bfloat16)
    y = (jnp.dot(xh, ah_ref[...], preferred_element_type=jnp.float32)
         + jnp.dot(xh, al_ref[...], preferred_element_type=jnp.float32)
         + jnp.dot(xl, ah_ref[...], preferred_element_type=jnp.float32))

    s = y[:, :E]
    h = y[:, E:] + p_ref[0, :H]

    mu = jnp.mean(h, axis=1, keepdims=True)
    var = jnp.mean((h - mu) * (h - mu), axis=1, keepdims=True)
    hn = (h - mu) / jnp.sqrt(var + 1e-5) * p_ref[1, :H] + p_ref[2, :H]
    hg = 0.5 * hn * (1.0 + jax.lax.erf(hn * 0.7071067811865476))

    cap_logit = jnp.sum(hg * p_ref[3, :H], axis=1, keepdims=True) + p_ref[5, 0]
    cap = jax.nn.sigmoid(cap_logit)

    g = (s + p_ref[4, :E]) * cap

    col = jax.lax.broadcasted_iota(jnp.int32, g.shape, 1)
    v1 = jnp.max(g, axis=1, keepdims=True)
    i1 = jnp.min(jnp.where(g == v1, col, E), axis=1, keepdims=True)
    m1 = col == i1
    gm = jnp.where(m1, -jnp.inf, g)
    v2 = jnp.max(gm, axis=1, keepdims=True)
    i2 = jnp.min(jnp.where(gm == v2, col, E), axis=1, keepdims=True)
    m2 = col == i2

    rw = jnp.where(m1, v1, jnp.where(m2, v2, 0.0))
    o_ref[...] = rw / (v1 + v2 + 1e-6)


@jax.jit
def _router(x, ah, al, params):
    return pl.pallas_call(
        _router_block,
        grid=(B // BM,),
        in_specs=[
            pl.BlockSpec((BM, D), lambda i: (i, 0)),
            pl.BlockSpec((D, 2 * E), lambda i: (0, 0)),
            pl.BlockSpec((D, 2 * E), lambda i: (0, 0)),
            pl.BlockSpec((8, 2 * E), lambda i: (0, 0)),
        ],
        out_specs=pl.BlockSpec((BM, E), lambda i: (i, 0)),
        out_shape=jax.ShapeDtypeStruct((B, E), jnp.float32),
    )(x, ah, al, params)


def kernel(x, gate_w, w1, b1, ln_g, ln_b, w2, b2, temperature, expert_usage):
    t = jnp.clip(temperature, 0.1, None)[0]
    lb = expert_usage / (jnp.sum(expert_usage) + 1e-6)
    a = jnp.concatenate([gate_w.T * t, w1.T], axis=1)
    ah = a.astype(jnp.bfloat16)
    al = (a - ah.astype(jnp.float32)).astype(jnp.bfloat16)
    bias = -0.1 * lb * t
    params = jnp.zeros((8, 2 * E), jnp.float32)
    params = params.at[0, :H].set(b1)
    params = params.at[1, :H].set(ln_g)
    params = params.at[2, :H].set(ln_b)
    params = params.at[3, :H].set(w2[0])
    params = params.at[4, :E].set(bias)
    params = params.at[5, 0].set(b2[0])
    return _router(x, ah, al, params)


# speedup vs baseline: 1.9873x; 1.9873x over previous
import jax
import jax.numpy as jnp
from jax.experimental import pallas as pl

B, D, E, H = 32768, 768, 64, 64
BM = 512


def _router_block(x_ref, a_ref, mred_ref, w2b_ref, p_ref, o_ref):
    xb = x_ref[...]
    y = jnp.dot(xb, a_ref[...], preferred_element_type=jnp.float32)

    s = y[:, :E]
    h = y[:, E:] + p_ref[0, :H]

    hh = jnp.concatenate([h, h * h], axis=1)
    m = jnp.dot(hh, mred_ref[...], preferred_element_type=jnp.float32)
    mu = m[:, :H]
    var = m[:, H:] - mu * mu
    hn = (h - mu) / jnp.sqrt(var + 1e-5) * p_ref[1, :H] + p_ref[2, :H]
    hg = 0.5 * hn * (1.0 + jax.lax.erf(hn * 0.7071067811865476))

    cap_logit = jnp.dot(hg, w2b_ref[...], preferred_element_type=jnp.float32)
    cap = jax.nn.sigmoid(cap_logit + p_ref[5, :E])

    g = (s + p_ref[4, :E]) * cap

    colf = jax.lax.broadcasted_iota(jnp.int32, g.shape, 1).astype(jnp.float32)
    v1 = jnp.max(g, axis=1, keepdims=True)
    i1 = jnp.min(jnp.where(g == v1, colf, float(E)), axis=1, keepdims=True)
    m1 = colf == i1
    gm = jnp.where(m1, -jnp.inf, g)
    v2 = jnp.max(gm, axis=1, keepdims=True)
    i2 = jnp.min(jnp.where(gm == v2, colf, float(E)), axis=1, keepdims=True)
    m2 = colf == i2

    rw = jnp.where(m1, v1, jnp.where(m2, v2, 0.0))
    o_ref[...] = rw / (v1 + v2 + 1e-6)


@jax.jit
def _router(x, a, mred, w2b, params):
    return pl.pallas_call(
        _router_block,
        grid=(B // BM,),
        in_specs=[
            pl.BlockSpec((BM, D), lambda i: (i, 0)),
            pl.BlockSpec((D, 2 * E), lambda i: (0, 0)),
            pl.BlockSpec((2 * H, 2 * H), lambda i: (0, 0)),
            pl.BlockSpec((H, E), lambda i: (0, 0)),
            pl.BlockSpec((8, 2 * E), lambda i: (0, 0)),
        ],
        out_specs=pl.BlockSpec((BM, E), lambda i: (i, 0)),
        out_shape=jax.ShapeDtypeStruct((B, E), jnp.float32),
    )(x, a, mred, w2b, params)


def kernel(x, gate_w, w1, b1, ln_g, ln_b, w2, b2, temperature, expert_usage):
    t = jnp.clip(temperature, 0.1, None)[0]
    lb = expert_usage / (jnp.sum(expert_usage) + 1e-6)
    a = jnp.concatenate([gate_w.T * t, w1.T], axis=1)
    bias = -0.1 * lb * t

    blk = jnp.ones((H, H), jnp.float32) / H
    z = jnp.zeros((H, H), jnp.float32)
    mred = jnp.block([[blk, z], [z, blk]])
    w2b = jnp.tile(w2[0][:, None], (1, E))

    params = jnp.zeros((8, 2 * E), jnp.float32)
    params = params.at[0, :H].set(b1)
    params = params.at[1, :H].set(ln_g)
    params = params.at[2, :H].set(ln_b)
    params = params.at[4, :E].set(bias)
    params = params.at[5, :E].set(b2[0])
    return _router(x, a, mred, w2b, params)


# speedup vs baseline: 2.5027x; 1.2593x over previous
import jax
import jax.numpy as jnp
from jax.experimental import pallas as pl

B, D, E, H = 32768, 768, 64, 64
BM = 1024


def _router_block(x_ref, a_ref, mred_ref, w2b_ref, p_ref, o_ref):
    xb = x_ref[...]
    y = jnp.dot(xb, a_ref[...], preferred_element_type=jnp.float32)

    s = y[:, :E]
    h = y[:, E:] + p_ref[0, :H]

    hh = jnp.concatenate([h, h * h], axis=1)
    m = jnp.dot(hh, mred_ref[...], preferred_element_type=jnp.float32)
    mu = m[:, :H]
    var = m[:, H:] - mu * mu
    hn = (h - mu) / jnp.sqrt(var + 1e-5) * p_ref[1, :H] + p_ref[2, :H]
    hg = 0.5 * hn * (1.0 + jax.lax.erf(hn * 0.7071067811865476))

    cap_logit = jnp.dot(hg, w2b_ref[...], preferred_element_type=jnp.float32)
    cap = jax.nn.sigmoid(cap_logit + p_ref[5, :E])

    g = (s + p_ref[4, :E]) * cap

    colf = jax.lax.broadcasted_iota(jnp.int32, g.shape, 1).astype(jnp.float32)
    v1 = jnp.max(g, axis=1, keepdims=True)
    i1 = jnp.min(jnp.where(g == v1, colf, float(E)), axis=1, keepdims=True)
    m1 = colf == i1
    gm = jnp.where(m1, -jnp.inf, g)
    v2 = jnp.max(gm, axis=1, keepdims=True)
    i2 = jnp.min(jnp.where(gm == v2, colf, float(E)), axis=1, keepdims=True)
    m2 = colf == i2

    rw = jnp.where(m1, v1, jnp.where(m2, v2, 0.0))
    o_ref[...] = rw / (v1 + v2 + 1e-6)


@jax.jit
def _router(x, a, mred, w2b, params):
    return pl.pallas_call(
        _router_block,
        grid=(B // BM,),
        in_specs=[
            pl.BlockSpec((BM, D), lambda i: (i, 0)),
            pl.BlockSpec((D, 2 * E), lambda i: (0, 0)),
            pl.BlockSpec((2 * H, 2 * H), lambda i: (0, 0)),
            pl.BlockSpec((H, E), lambda i: (0, 0)),
            pl.BlockSpec((8, 2 * E), lambda i: (0, 0)),
        ],
        out_specs=pl.BlockSpec((BM, E), lambda i: (i, 0)),
        out_shape=jax.ShapeDtypeStruct((B, E), jnp.float32),
    )(x, a, mred, w2b, params)


def kernel(x, gate_w, w1, b1, ln_g, ln_b, w2, b2, temperature, expert_usage):
    t = jnp.clip(temperature, 0.1, None)[0]
    lb = expert_usage / (jnp.sum(expert_usage) + 1e-6)
    a = jnp.concatenate([gate_w.T * t, w1.T], axis=1)
    bias = -0.1 * lb * t

    blk = jnp.ones((H, H), jnp.float32) / H
    z = jnp.zeros((H, H), jnp.float32)
    mred = jnp.block([[blk, z], [z, blk]])
    w2b = jnp.tile(w2[0][:, None], (1, E))

    params = jnp.zeros((8, 2 * E), jnp.float32)
    params = params.at[0, :H].set(b1)
    params = params.at[1, :H].set(ln_g)
    params = params.at[2, :H].set(ln_b)
    params = params.at[4, :E].set(bias)
    params = params.at[5, :E].set(b2[0])
    return _router(x, a, mred, w2b, params)


# speedup vs baseline: 2.7952x; 1.1169x over previous
import jax
import jax.numpy as jnp
from jax.experimental import pallas as pl

B, D, E, H = 32768, 768, 64, 64
BM = 2048


def _router_block(x_ref, a_ref, mred_ref, w2b_ref, p_ref, o_ref):
    xb = x_ref[...]
    y = jnp.dot(xb, a_ref[...], preferred_element_type=jnp.float32)

    s = y[:, :E]
    h = y[:, E:] + p_ref[0, :H]

    hh = jnp.concatenate([h, h * h], axis=1)
    m = jnp.dot(hh, mred_ref[...], preferred_element_type=jnp.float32)
    mu = m[:, :H]
    var = m[:, H:] - mu * mu
    hn = (h - mu) / jnp.sqrt(var + 1e-5) * p_ref[1, :H] + p_ref[2, :H]
    hg = 0.5 * hn * (1.0 + jax.lax.erf(hn * 0.7071067811865476))

    cap_logit = jnp.dot(hg, w2b_ref[...], preferred_element_type=jnp.float32)
    cap = jax.nn.sigmoid(cap_logit + p_ref[5, :E])

    g = (s + p_ref[4, :E]) * cap

    colf = jax.lax.broadcasted_iota(jnp.int32, g.shape, 1).astype(jnp.float32)
    v1 = jnp.max(g, axis=1, keepdims=True)
    i1 = jnp.min(jnp.where(g == v1, colf, float(E)), axis=1, keepdims=True)
    m1 = colf == i1
    gm = jnp.where(m1, -jnp.inf, g)
    v2 = jnp.max(gm, axis=1, keepdims=True)
    i2 = jnp.min(jnp.where(gm == v2, colf, float(E)), axis=1, keepdims=True)
    m2 = colf == i2

    rw = jnp.where(m1, v1, jnp.where(m2, v2, 0.0))
    o_ref[...] = rw / (v1 + v2 + 1e-6)


@jax.jit
def _router(x, a, mred, w2b, params):
    return pl.pallas_call(
        _router_block,
        grid=(B // BM,),
        in_specs=[
            pl.BlockSpec((BM, D), lambda i: (i, 0)),
            pl.BlockSpec((D, 2 * E), lambda i: (0, 0)),
            pl.BlockSpec((2 * H, 2 * H), lambda i: (0, 0)),
            pl.BlockSpec((H, E), lambda i: (0, 0)),
            pl.BlockSpec((8, 2 * E), lambda i: (0, 0)),
        ],
        out_specs=pl.BlockSpec((BM, E), lambda i: (i, 0)),
        out_shape=jax.ShapeDtypeStruct((B, E), jnp.float32),
    )(x, a, mred, w2b, params)


def kernel(x, gate_w, w1, b1, ln_g, ln_b, w2, b2, temperature, expert_usage):
    t = jnp.clip(temperature, 0.1, None)[0]
    lb = expert_usage / (jnp.sum(expert_usage) + 1e-6)
    a = jnp.concatenate([gate_w.T * t, w1.T], axis=1)
    bias = -0.1 * lb * t

    blk = jnp.ones((H, H), jnp.float32) / H
    z = jnp.zeros((H, H), jnp.float32)
    mred = jnp.block([[blk, z], [z, blk]])
    w2b = jnp.tile(w2[0][:, None], (1, E))

    params = jnp.zeros((8, 2 * E), jnp.float32)
    params = params.at[0, :H].set(b1)
    params = params.at[1, :H].set(ln_g)
    params = params.at[2, :H].set(ln_b)
    params = params.at[4, :E].set(bias)
    params = params.at[5, :E].set(b2[0])
    return _router(x, a, mred, w2b, params)


# speedup vs baseline: 2.9825x; 1.0670x over previous
import jax
import jax.numpy as jnp
from jax.experimental import pallas as pl

B, D, E, H = 32768, 768, 64, 64
BM = 4096


def _router_block(x_ref, a_ref, mred_ref, w2b_ref, p_ref, o_ref):
    xb = x_ref[...]
    y = jnp.dot(xb, a_ref[...], preferred_element_type=jnp.float32)

    s = y[:, :E]
    h = y[:, E:] + p_ref[0, :H]

    hh = jnp.concatenate([h, h * h], axis=1)
    m = jnp.dot(hh, mred_ref[...], preferred_element_type=jnp.float32)
    mu = m[:, :H]
    var = m[:, H:] - mu * mu
    hn = (h - mu) / jnp.sqrt(var + 1e-5) * p_ref[1, :H] + p_ref[2, :H]
    hg = 0.5 * hn * (1.0 + jax.lax.erf(hn * 0.7071067811865476))

    cap_logit = jnp.dot(hg, w2b_ref[...], preferred_element_type=jnp.float32)
    cap = jax.nn.sigmoid(cap_logit + p_ref[5, :E])

    g = (s + p_ref[4, :E]) * cap

    colf = jax.lax.broadcasted_iota(jnp.int32, g.shape, 1).astype(jnp.float32)
    v1 = jnp.max(g, axis=1, keepdims=True)
    i1 = jnp.min(jnp.where(g == v1, colf, float(E)), axis=1, keepdims=True)
    m1 = colf == i1
    gm = jnp.where(m1, -jnp.inf, g)
    v2 = jnp.max(gm, axis=1, keepdims=True)
    i2 = jnp.min(jnp.where(gm == v2, colf, float(E)), axis=1, keepdims=True)
    m2 = colf == i2

    rw = jnp.where(m1, v1, jnp.where(m2, v2, 0.0))
    o_ref[...] = rw / (v1 + v2 + 1e-6)


@jax.jit
def _router(x, a, mred, w2b, params):
    return pl.pallas_call(
        _router_block,
        grid=(B // BM,),
        in_specs=[
            pl.BlockSpec((BM, D), lambda i: (i, 0)),
            pl.BlockSpec((D, 2 * E), lambda i: (0, 0)),
            pl.BlockSpec((2 * H, 2 * H), lambda i: (0, 0)),
            pl.BlockSpec((H, E), lambda i: (0, 0)),
            pl.BlockSpec((8, 2 * E), lambda i: (0, 0)),
        ],
        out_specs=pl.BlockSpec((BM, E), lambda i: (i, 0)),
        out_shape=jax.ShapeDtypeStruct((B, E), jnp.float32),
    )(x, a, mred, w2b, params)


def kernel(x, gate_w, w1, b1, ln_g, ln_b, w2, b2, temperature, expert_usage):
    t = jnp.clip(temperature, 0.1, None)[0]
    lb = expert_usage / (jnp.sum(expert_usage) + 1e-6)
    a = jnp.concatenate([gate_w.T * t, w1.T], axis=1)
    bias = -0.1 * lb * t

    blk = jnp.ones((H, H), jnp.float32) / H
    z = jnp.zeros((H, H), jnp.float32)
    mred = jnp.block([[blk, z], [z, blk]])
    w2b = jnp.tile(w2[0][:, None], (1, E))

    params = jnp.zeros((8, 2 * E), jnp.float32)
    params = params.at[0, :H].set(b1)
    params = params.at[1, :H].set(ln_g)
    params = params.at[2, :H].set(ln_b)
    params = params.at[4, :E].set(bias)
    params = params.at[5, :E].set(b2[0])
    return _router(x, a, mred, w2b, params)


# speedup vs baseline: 3.3732x; 1.1310x over previous
import jax
import jax.numpy as jnp
from jax.experimental import pallas as pl

B, D, E, H = 32768, 768, 64, 64
BM = 4096


def _router_block(x_ref, a_ref, mred_ref, w2b_ref, p_ref, o_ref):
    xb = x_ref[...]
    y = jnp.dot(xb, a_ref[...], preferred_element_type=jnp.float32)

    s = y[:, :E]
    h = y[:, E:] + p_ref[0, :H]

    hh = jnp.concatenate([h, h * h], axis=1)
    m = jnp.dot(hh, mred_ref[...], preferred_element_type=jnp.float32)
    mu = m[:, :H]
    var = m[:, H:] - mu * mu
    hn = (h - mu) * jax.lax.rsqrt(var + 1e-5) * p_ref[1, :H] + p_ref[2, :H]
    hg = 0.5 * hn * (1.0 + jax.lax.erf(hn * 0.7071067811865476))

    cap_logit = jnp.dot(hg, w2b_ref[...], preferred_element_type=jnp.float32)
    cap = jax.nn.sigmoid(cap_logit + p_ref[5, :E])

    g = (s + p_ref[4, :E]) * cap

    v1 = jnp.max(g, axis=1, keepdims=True)
    m1 = g == v1
    gm = jnp.where(m1, -jnp.inf, g)
    v2 = jnp.max(gm, axis=1, keepdims=True)
    m2 = gm == v2

    rw = jnp.where(m1, v1, jnp.where(m2, v2, 0.0))
    o_ref[...] = rw / (v1 + v2 + 1e-6)


@jax.jit
def _router(x, a, mred, w2b, params):
    return pl.pallas_call(
        _router_block,
        grid=(B // BM,),
        in_specs=[
            pl.BlockSpec((BM, D), lambda i: (i, 0)),
            pl.BlockSpec((D, 2 * E), lambda i: (0, 0)),
            pl.BlockSpec((2 * H, 2 * H), lambda i: (0, 0)),
            pl.BlockSpec((H, E), lambda i: (0, 0)),
            pl.BlockSpec((8, 2 * E), lambda i: (0, 0)),
        ],
        out_specs=pl.BlockSpec((BM, E), lambda i: (i, 0)),
        out_shape=jax.ShapeDtypeStruct((B, E), jnp.float32),
    )(x, a, mred, w2b, params)


def kernel(x, gate_w, w1, b1, ln_g, ln_b, w2, b2, temperature, expert_usage):
    t = jnp.clip(temperature, 0.1, None)[0]
    lb = expert_usage / (jnp.sum(expert_usage) + 1e-6)
    a = jnp.concatenate([gate_w.T * t, w1.T], axis=1)
    bias = -0.1 * lb * t

    blk = jnp.ones((H, H), jnp.float32) / H
    z = jnp.zeros((H, H), jnp.float32)
    mred = jnp.block([[blk, z], [z, blk]])
    w2b = jnp.tile(w2[0][:, None], (1, E))

    params = jnp.zeros((8, 2 * E), jnp.float32)
    params = params.at[0, :H].set(b1)
    params = params.at[1, :H].set(ln_g)
    params = params.at[2, :H].set(ln_b)
    params = params.at[4, :E].set(bias)
    params = params.at[5, :E].set(b2[0])
    return _router(x, a, mred, w2b, params)
